# Initial kernel scaffold; baseline (speedup 1.0000x reference)
import jax
import jax.numpy as jnp
from jax.experimental import pallas as pl


def kernel(xyz, points, affine_alpha, affine_beta):
    raise NotImplementedError("write your pallas kernel here")


# speedup vs baseline: 14.9203x; 14.9203x over previous
import functools

import jax
import jax.numpy as jnp
from jax import lax
from jax.experimental import pallas as pl
from jax.experimental.pallas import tpu as pltpu
from jax.experimental.pallas import tpu_sc as plsc

_B, _N = 8, 8192
_D = 128
_S = 2048
_K = 32
_CH = _D + 3
_SQ = 64
_NT = _S // _SQ
_GC = 256
_NTOT = _B * _S * _K
_NF = _B * _S


def _fps_body(xp_ref, nx_ref, nxt_ref, fidx_ref):
    x = xp_ref[0]
    y = xp_ref[1]
    z = xp_ref[2]
    iota = lax.broadcasted_iota(jnp.int32, (_B, _N), 1)
    boff = lax.broadcasted_iota(jnp.int32, (_B, 1), 0) * _N
    i128 = lax.broadcasted_iota(jnp.int32, (_B, 128), 1)

    def step(t, carry):
        dist, far, bfi, bfx, bfy, bfz = carry
        mask = iota == far
        cx = jnp.sum(jnp.where(mask, x, 0.0), axis=1, keepdims=True)
        cy = jnp.sum(jnp.where(mask, y, 0.0), axis=1, keepdims=True)
        cz = jnp.sum(jnp.where(mask, z, 0.0), axis=1, keepdims=True)
        sel = i128 == (t % 128)
        bfi = jnp.where(sel, far + boff, bfi)
        bfx = jnp.where(sel, cx, bfx)
        bfy = jnp.where(sel, cy, bfy)
        bfz = jnp.where(sel, cz, bfz)

        @pl.when(t % 128 == 127)
        def _():
            t0 = pl.multiple_of((t // 128) * 128, 128)
            fidx_ref[:, pl.ds(t0, 128)] = bfi
            c3 = jnp.stack([bfx, bfy, bfz], axis=1)
            nxt_ref[:, :, pl.ds(t0, 128)] = c3
            nx_ref[:, pl.ds(t0, 128), :] = jnp.transpose(c3, (0, 2, 1))

        dx = x - cx
        dy = y - cy
        dz = z - cz
        d = (dx * dx + dy * dy) + dz * dz
        dist = jnp.minimum(dist, d)
        m = jnp.max(dist, axis=1, keepdims=True)
        far = jnp.min(jnp.where(dist == m, iota, _N), axis=1, keepdims=True)
        return dist, far, bfi, bfx, bfy, bfz

    dist0 = jnp.full((_B, _N), 1e10, jnp.float32)
    far0 = jnp.zeros((_B, 1), jnp.int32)
    zf = jnp.zeros((_B, 128), jnp.float32)
    zi = jnp.zeros((_B, 128), jnp.int32)
    lax.fori_loop(0, _S, step, (dist0, far0, zi, zf, zf, zf))


def _fps(xyz):
    xp = jnp.transpose(xyz, (2, 0, 1))
    return pl.pallas_call(
        _fps_body,
        out_shape=(
            jax.ShapeDtypeStruct((_B, _S, 3), jnp.float32),
            jax.ShapeDtypeStruct((_B, 3, _S), jnp.float32),
            jax.ShapeDtypeStruct((_B, _S), jnp.int32),
        ),
    )(xp)


def _ce(av, ai, bv, bi):
    c = bv < av
    return (jnp.where(c, bv, av), jnp.where(c, bi, ai),
            jnp.where(c, av, bv), jnp.where(c, ai, bi))


def _rev1(x):
    g0, m0 = x.shape[0], x.shape[1]
    tail = x.shape[2:]
    g, m = g0, m0
    while m > 1:
        x = x.reshape(g, 2, m // 2, *tail)
        x = jnp.concatenate([x[:, 1:2], x[:, 0:1]], axis=1)
        x = x.reshape(g * 2, m // 2, *tail)
        g, m = g * 2, m // 2
    return x.reshape(g0, m0, *tail)


def _clean(v, i):
    g, m = v.shape[0], v.shape[1]
    tail = v.shape[2:]
    s = m // 2
    while s >= 1:
        vr = v.reshape(g, m // (2 * s), 2, s, *tail)
        ir = i.reshape(g, m // (2 * s), 2, s, *tail)
        mn, mni, mx, mxi = _ce(vr[:, :, 0], ir[:, :, 0], vr[:, :, 1], ir[:, :, 1])
        v = jnp.stack([mn, mx], axis=2).reshape(g, m, *tail)
        i = jnp.stack([mni, mxi], axis=2).reshape(g, m, *tail)
        s //= 2
    return v, i


def _merge_pairs(v, i, keep):
    nl, m = v.shape[0], v.shape[1]
    tail = v.shape[2:]
    vv = v.reshape(nl // 2, 2, m, *tail)
    ii = i.reshape(nl // 2, 2, m, *tail)
    a, ai = vv[:, 0], ii[:, 0]
    b, bi = _rev1(vv[:, 1]), _rev1(ii[:, 1])
    mn, mni, mx, mxi = _ce(a, ai, b, bi)
    if keep == m:
        return _clean(mn, mni)
    mnc, mnci = _clean(mn, mni)
    mxc, mxci = _clean(mx, mxi)
    return (jnp.concatenate([mnc, mxc], axis=1),
            jnp.concatenate([mnci, mxci], axis=1))


def _merge_lanes(v, i, shift, keep):
    pv = jnp.roll(v, -shift, axis=2)
    pi = jnp.roll(i, -shift, axis=2)
    m = v.shape[0]
    mn, mni, mx, mxi = _ce(v, i, _rev1(pv[None])[0], _rev1(pi[None])[0])
    if keep == m:
        r = _clean(mn[None], mni[None])
        return r[0][0], r[1][0]
    a = _clean(mn[None], mni[None])
    b = _clean(mx[None], mxi[None])
    return (jnp.concatenate([a[0][0], b[0][0]], axis=0),
            jnp.concatenate([a[1][0], b[1][0]], axis=0))


def _knn_body(xp_ref, sq_ref, q_ref, idx_ref):
    b = pl.program_id(0)
    q = q_ref[0, 0]
    q8 = jnp.concatenate([q, jnp.zeros((5, _SQ), jnp.float32)], axis=0)
    qt = jnp.transpose(q8, (1, 0))
    p = xp_ref[0]
    qp = lax.dot_general(qt, p, (((1,), (0,)), ((), ())),
                         preferred_element_type=jnp.float32)
    q2 = jnp.sum(qt[:, :3] * qt[:, :3], axis=1, keepdims=True)
    dflat = -2.0 * qp + q2
    d = jnp.transpose(dflat.reshape(_SQ, _N // 128, 128), (1, 0, 2))
    d = d + sq_ref[0][:, None, :]
    i0 = (lax.broadcasted_iota(jnp.int32, d.shape, 0) * 128
          + lax.broadcasted_iota(jnp.int32, d.shape, 2))

    v, i = d[:, None], i0[:, None]
    for m in (1, 2, 4):
        v, i = _merge_pairs(v, i, keep=2 * m)
    for _ in range(3):
        v, i = _merge_pairs(v, i, keep=8)
    v, i = v[0], i[0]

    v, i = _merge_lanes(v, i, 1, keep=16)
    v, i = _merge_lanes(v, i, 2, keep=32)
    for j in range(2, 7):
        v, i = _merge_lanes(v, i, 1 << j, keep=32)

    idx_ref[0, 0] = i[:, :, 0] + b * _N


def _knn(xq, sq, nxq):
    grid = (_B, _NT)
    return pl.pallas_call(
        _knn_body,
        grid=grid,
        in_specs=[
            pl.BlockSpec((1, 8, _N), lambda b, t: (b, 0, 0)),
            pl.BlockSpec((1, _N // 128, 128), lambda b, t: (b, 0, 0)),
            pl.BlockSpec((1, 1, 3, _SQ), lambda b, t: (b, t, 0, 0)),
        ],
        out_specs=pl.BlockSpec((1, 1, _K, _SQ), lambda b, t: (b, t, 0, 0)),
        out_shape=jax.ShapeDtypeStruct((_B, _NT, _K, _SQ), jnp.int32),
    )(xq, sq, nxq)


def _make_gather():
    mesh = plsc.VectorSubcoreMesh(core_axis_name="c", subcore_axis_name="s")
    rows_w = _NTOT // 32
    rows_f = _NF // 32
    xl = _N * 3

    @functools.partial(
        pl.kernel, mesh=mesh,
        out_type=(
            jax.ShapeDtypeStruct((_NTOT, _D), jnp.float32),
            jax.ShapeDtypeStruct((_NTOT * 16,), jnp.float32),
            jax.ShapeDtypeStruct((_NF, _D), jnp.float32),
        ),
        scratch_types=[
            pltpu.VMEM((_GC,), jnp.int32),
            pltpu.VMEM((_GC, _D), jnp.float32),
            pltpu.VMEM((_GC * 16,), jnp.float32),
            pltpu.VMEM((xl + 16,), jnp.float32),
            pltpu.SemaphoreType.DMA,
        ],
    )
    def gth(t1, xyzf, idx, fidx, g1, g2f, g3, idx_v, r1_v, r2f_v, xyz_v, s1):
        wid = lax.axis_index("s") * 2 + lax.axis_index("c")
        b = wid // 4
        bn = b * _N
        pltpu.sync_copy(xyzf.at[pl.ds(b * xl, xl)], xyz_v.at[pl.ds(0, xl)])
        zero16 = jnp.zeros((16,), jnp.float32)
        iota16 = lax.broadcasted_iota(jnp.int32, (16,), 0)

        def zinit(j, _):
            r2f_v[pl.ds(j * 16, 16)] = zero16
            return 0

        lax.fori_loop(0, _GC, zinit, 0)

        def it(j, _):
            base = wid * rows_w + j * _GC
            pltpu.sync_copy(idx.at[pl.ds(base, _GC)], idx_v)
            c1 = pltpu.async_copy(t1.at[idx_v], r1_v, s1)

            def grp(g, _):
                jv = idx_v[pl.ds(g * 16, 16)] - bn
                for k in range(16):
                    a = jv[k] * 3
                    xv = xyz_v[pl.ds(a, 16)]
                    r2f_v[pl.ds((g * 16 + k) * 16, 16)] = jnp.where(
                        iota16 < 3, xv, 0.0)
                return 0

            lax.fori_loop(0, _GC // 16, grp, 0)
            c1.wait()
            pltpu.sync_copy(r1_v, g1.at[pl.ds(base, _GC)])
            pltpu.sync_copy(r2f_v, g2f.at[pl.ds(base * 16, _GC * 16)])
            return 0

        lax.fori_loop(0, rows_w // _GC, it, 0)

        def it2(j, _):
            base = wid * rows_f + j * _GC
            pltpu.sync_copy(fidx.at[pl.ds(base, _GC)], idx_v)
            pltpu.async_copy(t1.at[idx_v], r1_v, s1).wait()
            pltpu.sync_copy(r1_v, g3.at[pl.ds(base, _GC)])
            return 0

        lax.fori_loop(0, rows_f // _GC, it2, 0)

    return gth


_gather = _make_gather()


def _stats_body(g1_ref, g2_ref, m1_ref, m2_ref, var_ref, acc_ref):
    t = pl.program_id(1)
    g1 = g1_ref[0, 0]
    g2 = g2_ref[0, 0]
    mean1 = jnp.mean(g1, axis=0)
    mean2 = jnp.mean(g2, axis=0)
    m1_ref[0, 0] = mean1
    m2_ref[0, 0] = mean2
    dev1 = g1 - mean1[None]
    dev2 = g2 - mean2[None]
    ssq = jnp.sum(dev1 * dev1) + jnp.sum(dev2 * dev2)

    @pl.when(t == 0)
    def _():
        acc_ref[...] = jnp.zeros((1, 1), jnp.float32)

    acc_ref[...] += ssq.reshape(1, 1)

    @pl.when(t == _NT - 1)
    def _():
        var_ref[...] = jnp.broadcast_to(acc_ref[0, 0], (1, 1, 128))


def _stats(g1, g2):
    grid = (_B, _NT)
    return pl.pallas_call(
        _stats_body,
        grid=grid,
        in_specs=[
            pl.BlockSpec((1, 1, _K, _SQ, _D), lambda b, t: (b, t, 0, 0, 0)),
            pl.BlockSpec((1, 1, _K, _SQ, 16), lambda b, t: (b, t, 0, 0, 0)),
        ],
        out_specs=(
            pl.BlockSpec((1, 1, _SQ, _D), lambda b, t: (b, t, 0, 0)),
            pl.BlockSpec((1, 1, _SQ, 16), lambda b, t: (b, t, 0, 0)),
            pl.BlockSpec((1, 1, 128), lambda b, t: (b, 0, 0)),
        ),
        out_shape=(
            jax.ShapeDtypeStruct((_B, _NT, _SQ, _D), jnp.float32),
            jax.ShapeDtypeStruct((_B, _NT, _SQ, 16), jnp.float32),
            jax.ShapeDtypeStruct((_B, 1, 128), jnp.float32),
        ),
        scratch_shapes=[pltpu.VMEM((1, 1), jnp.float32)],
    )(g1, g2)


def _final_body(g1_ref, g2_ref, g3_ref, m1_ref, m2_ref, var_ref,
                a1_ref, a2_ref, b1_ref, b2_ref, out_ref):
    n1 = _S * _K * _CH - 1
    std = jnp.sqrt(var_ref[0, 0, 0] / n1)
    den = std + 1e-5
    g1 = g1_ref[0, 0]
    g2 = g2_ref[0, 0]
    m1 = m1_ref[0, 0]
    m2 = m2_ref[0, 0]
    p1 = a1_ref[0][None, None, :] * ((g1 - m1[None]) / den) + b1_ref[0][None, None, :]
    p2 = a2_ref[0][None, None, :] * ((g2 - m2[None]) / den) + b2_ref[0][None, None, :]
    p1 = jnp.transpose(p1, (1, 0, 2))
    p2 = jnp.transpose(p2, (1, 0, 2))
    g3 = g3_ref[0, 0]
    p3 = jnp.broadcast_to(g3[:, None, :], (_SQ, _K, _D))
    out_ref[0, 0] = jnp.concatenate([p1, p2[:, :, :3], p3], axis=-1)


def _final(g1, g2, g3, m1, m2, var, a1, a2, b1, b2):
    grid = (_B, _NT)
    return pl.pallas_call(
        _final_body,
        grid=grid,
        in_specs=[
            pl.BlockSpec((1, 1, _K, _SQ, _D), lambda b, t: (b, t, 0, 0, 0)),
            pl.BlockSpec((1, 1, _K, _SQ, 16), lambda b, t: (b, t, 0, 0, 0)),
            pl.BlockSpec((1, 1, _SQ, _D), lambda b, t: (b, t, 0, 0)),
            pl.BlockSpec((1, 1, _SQ, _D), lambda b, t: (b, t, 0, 0)),
            pl.BlockSpec((1, 1, _SQ, 16), lambda b, t: (b, t, 0, 0)),
            pl.BlockSpec((1, 1, 128), lambda b, t: (b, 0, 0)),
            pl.BlockSpec((1, _D), lambda b, t: (0, 0)),
            pl.BlockSpec((1, 16), lambda b, t: (0, 0)),
            pl.BlockSpec((1, _D), lambda b, t: (0, 0)),
            pl.BlockSpec((1, 16), lambda b, t: (0, 0)),
        ],
        out_specs=pl.BlockSpec((1, 1, _SQ, _K, _D * 2 + 3),
                               lambda b, t: (b, t, 0, 0, 0)),
        out_shape=jax.ShapeDtypeStruct((_B, _NT, _SQ, _K, _D * 2 + 3), jnp.float32),
    )(g1, g2, g3, m1, m2, var, a1, a2, b1, b2)


def kernel(xyz, points, affine_alpha, affine_beta):
    new_xyz, nxt, fidx = _fps(xyz)

    xq = jnp.pad(jnp.transpose(xyz, (2, 0, 1)), ((0, 5), (0, 0), (0, 0)))
    xq = jnp.transpose(xq, (1, 0, 2))
    sq = jnp.sum(xyz * xyz, axis=-1).reshape(_B, _N // 128, 128)
    nxq = jnp.transpose(nxt.reshape(_B, 3, _NT, _SQ), (0, 2, 1, 3))
    idxg = _knn(xq, sq, nxq)

    t1 = points.reshape(_B * _N, _D)
    xyzf = xyz.reshape(_B * _N * 3)
    g1, g2, g3 = _gather(t1, xyzf, idxg.reshape(_NTOT), fidx.reshape(_NF))

    g1v = g1.reshape(_B, _NT, _K, _SQ, _D)
    g2v = g2.reshape(_B, _NT, _K, _SQ, 16)
    m1, m2, var = _stats(g1v, g2v)

    al = affine_alpha.reshape(_CH)
    be = affine_beta.reshape(_CH)
    a1 = al[:_D].reshape(1, _D)
    a2 = jnp.pad(al[_D:], (0, 13)).reshape(1, 16)
    b1 = be[:_D].reshape(1, _D)
    b2 = jnp.pad(be[_D:], (0, 13)).reshape(1, 16)

    g3v = g3.reshape(_B, _NT, _SQ, _D)
    out = _final(g1v, g2v, g3v, m1, m2, var, a1, a2, b1, b2)
    return new_xyz, out.reshape(_B, _S, _K, _D * 2 + 3)


# speedup vs baseline: 17.8306x; 1.1951x over previous
import functools

import jax
import jax.numpy as jnp
from jax import lax
from jax.experimental import pallas as pl
from jax.experimental.pallas import tpu as pltpu
from jax.experimental.pallas import tpu_sc as plsc

_B, _N = 8, 8192
_D = 128
_S = 2048
_K = 32
_CH = _D + 3
_SQ = 64
_NT = _S // _SQ
_GC = 256
_NTOT = _B * _S * _K
_NF = _B * _S


def _fps_body(xp_ref, nx_ref, fidx_ref):
    x = xp_ref[0]
    y = xp_ref[1]
    z = xp_ref[2]
    iota = lax.broadcasted_iota(jnp.int32, (_B, _N), 1)
    boff = lax.broadcasted_iota(jnp.int32, (_B, 1), 0) * _N
    i128 = lax.broadcasted_iota(jnp.int32, (_B, 128), 1)

    def step(t, carry):
        dist, far, bfi, bfx, bfy, bfz = carry
        mask = iota == far
        cx = jnp.sum(jnp.where(mask, x, 0.0), axis=1, keepdims=True)
        cy = jnp.sum(jnp.where(mask, y, 0.0), axis=1, keepdims=True)
        cz = jnp.sum(jnp.where(mask, z, 0.0), axis=1, keepdims=True)
        sel = i128 == (t % 128)
        bfi = jnp.where(sel, far + boff, bfi)
        bfx = jnp.where(sel, cx, bfx)
        bfy = jnp.where(sel, cy, bfy)
        bfz = jnp.where(sel, cz, bfz)

        @pl.when(t % 128 == 127)
        def _():
            t0 = pl.multiple_of((t // 128) * 128, 128)
            fidx_ref[:, pl.ds(t0, 128)] = bfi
            c3 = jnp.stack([bfx, bfy, bfz], axis=1)
            nx_ref[:, pl.ds(t0, 128), :] = jnp.transpose(c3, (0, 2, 1))

        dx = x - cx
        dy = y - cy
        dz = z - cz
        d = (dx * dx + dy * dy) + dz * dz
        dist = jnp.minimum(dist, d)
        m = jnp.max(dist, axis=1, keepdims=True)
        far = jnp.min(jnp.where(dist == m, iota, _N), axis=1, keepdims=True)
        return dist, far, bfi, bfx, bfy, bfz

    dist0 = jnp.full((_B, _N), 1e10, jnp.float32)
    far0 = jnp.zeros((_B, 1), jnp.int32)
    zf = jnp.zeros((_B, 128), jnp.float32)
    zi = jnp.zeros((_B, 128), jnp.int32)
    lax.fori_loop(0, _S, step, (dist0, far0, zi, zf, zf, zf))


def _fps(xyz):
    xp = jnp.transpose(xyz, (2, 0, 1))
    return pl.pallas_call(
        _fps_body,
        out_shape=(
            jax.ShapeDtypeStruct((_B, _S, 3), jnp.float32),
            jax.ShapeDtypeStruct((_B, _S), jnp.int32),
        ),
    )(xp)


def _cemm(a, b):
    c = b[0] < a[0]
    return ((jnp.where(c, b[0], a[0]), jnp.where(c, b[1], a[1])),
            (jnp.where(c, a[0], b[0]), jnp.where(c, a[1], b[1])))


def _cemin(a, b):
    c = b[0] < a[0]
    return (jnp.where(c, b[0], a[0]), jnp.where(c, b[1], a[1]))


def _clean_l(lst):
    m = len(lst)
    s = m // 2
    while s >= 1:
        for base in range(0, m, 2 * s):
            for r in range(s):
                lst[base + r], lst[base + r + s] = _cemm(
                    lst[base + r], lst[base + r + s])
        s //= 2
    return lst


def _merge_l(a, b, keep):
    m = len(a)
    if keep == m:
        mn = [_cemin(a[j], b[m - 1 - j]) for j in range(m)]
        return _clean_l(mn)
    mn, mx = [], []
    for j in range(m):
        lo, hi = _cemm(a[j], b[m - 1 - j])
        mn.append(lo)
        mx.append(hi)
    return _clean_l(mn) + _clean_l(mx)


def _knn_body(xp_ref, sq_ref, q_ref, idx_ref):
    b = pl.program_id(0)
    qt = q_ref[0, 0]
    p = xp_ref[0]
    qp = lax.dot_general(qt, p, (((1,), (0,)), ((), ())),
                         preferred_element_type=jnp.float32)
    q2 = jnp.sum(qt[:, :3] * qt[:, :3], axis=1, keepdims=True)
    d0 = (-2.0 * qp + q2) + sq_ref[0]
    i0 = lax.broadcasted_iota(jnp.int32, (_SQ, _N), 1)

    cur = [(d0, i0)]
    w = _N
    for keep in (2, 4, 8, 8, 8, 8):
        h = w // 2
        a = [(v[:, :h], i[:, :h]) for v, i in cur]
        bb = [(v[:, h:], i[:, h:]) for v, i in cur]
        cur = _merge_l(a, bb, keep)
        w = h

    for shift, keep in ((1, 16), (2, 16), (4, 16), (8, 32),
                        (16, 32), (32, 32), (64, 32)):
        part = [(jnp.roll(v, -shift, axis=1), jnp.roll(i, -shift, axis=1))
                for v, i in cur]
        cur = _merge_l(cur, part, keep)

    res = jnp.concatenate([i[:, 0:1] for v, i in cur], axis=1)
    idx_ref[0, 0] = res + b * _N


def _knn(xq, sq, nxq):
    grid = (_B, _NT)
    return pl.pallas_call(
        _knn_body,
        grid=grid,
        in_specs=[
            pl.BlockSpec((1, 8, _N), lambda b, t: (b, 0, 0)),
            pl.BlockSpec((1, 1, _N), lambda b, t: (b, 0, 0)),
            pl.BlockSpec((1, 1, _SQ, 8), lambda b, t: (b, t, 0, 0)),
        ],
        out_specs=pl.BlockSpec((1, 1, _SQ, _K), lambda b, t: (b, t, 0, 0)),
        out_shape=jax.ShapeDtypeStruct((_B, _NT, _SQ, _K), jnp.int32),
    )(xq, sq, nxq)


def _make_gather():
    mesh = plsc.VectorSubcoreMesh(core_axis_name="c", subcore_axis_name="s")
    rows_w = _NTOT // 32
    rows_f = _NF // 32
    xl = _N * 3

    @functools.partial(
        pl.kernel, mesh=mesh,
        out_type=(
            jax.ShapeDtypeStruct((_NTOT, _D), jnp.float32),
            jax.ShapeDtypeStruct((_NTOT * 16,), jnp.float32),
            jax.ShapeDtypeStruct((_NF, _D), jnp.float32),
        ),
        scratch_types=[
            pltpu.VMEM((_GC,), jnp.int32),
            pltpu.VMEM((_GC, _D), jnp.float32),
            pltpu.VMEM((_GC * 16,), jnp.float32),
            pltpu.VMEM((xl + 16,), jnp.float32),
            pltpu.SemaphoreType.DMA,
        ],
    )
    def gth(t1, xyzf, idx, fidx, g1, g2f, g3, idx_v, r1_v, r2f_v, xyz_v, s1):
        wid = lax.axis_index("s") * 2 + lax.axis_index("c")
        b = wid // 4
        bn = b * _N
        pltpu.sync_copy(xyzf.at[pl.ds(b * xl, xl)], xyz_v.at[pl.ds(0, xl)])
        zero16 = jnp.zeros((16,), jnp.float32)
        iota16 = lax.broadcasted_iota(jnp.int32, (16,), 0)

        def zinit(j, _):
            r2f_v[pl.ds(j * 16, 16)] = zero16
            return 0

        lax.fori_loop(0, _GC, zinit, 0)

        def it(j, _):
            base = wid * rows_w + j * _GC
            pltpu.sync_copy(idx.at[pl.ds(base, _GC)], idx_v)
            c1 = pltpu.async_copy(t1.at[idx_v], r1_v, s1)

            def grp(g, _):
                jv = idx_v[pl.ds(g * 16, 16)] - bn
                for k in range(16):
                    a = jv[k] * 3
                    xv = xyz_v[pl.ds(a, 16)]
                    r2f_v[pl.ds((g * 16 + k) * 16, 16)] = jnp.where(
                        iota16 < 3, xv, 0.0)
                return 0

            lax.fori_loop(0, _GC // 16, grp, 0)
            c1.wait()
            pltpu.sync_copy(r1_v, g1.at[pl.ds(base, _GC)])
            pltpu.sync_copy(r2f_v, g2f.at[pl.ds(base * 16, _GC * 16)])
            return 0

        lax.fori_loop(0, rows_w // _GC, it, 0)

        def it2(j, _):
            base = wid * rows_f + j * _GC
            pltpu.sync_copy(fidx.at[pl.ds(base, _GC)], idx_v)
            pltpu.async_copy(t1.at[idx_v], r1_v, s1).wait()
            pltpu.sync_copy(r1_v, g3.at[pl.ds(base, _GC)])
            return 0

        lax.fori_loop(0, rows_f // _GC, it2, 0)

    return gth


_gather = _make_gather()


def _stats_body(g1_ref, g2_ref, m1_ref, m2_ref, var_ref, acc_ref):
    t = pl.program_id(1)
    g1 = g1_ref[0, 0]
    g2 = g2_ref[0, 0]
    mean1 = jnp.mean(g1, axis=1)
    mean2 = jnp.mean(g2, axis=1)
    m1_ref[0, 0] = mean1
    m2_ref[0, 0] = mean2
    dev1 = g1 - mean1[:, None]
    dev2 = g2 - mean2[:, None]
    ssq = jnp.sum(dev1 * dev1) + jnp.sum(dev2 * dev2)

    @pl.when(t == 0)
    def _():
        acc_ref[...] = jnp.zeros((1, 1), jnp.float32)

    acc_ref[...] += ssq.reshape(1, 1)

    @pl.when(t == _NT - 1)
    def _():
        var_ref[...] = jnp.broadcast_to(acc_ref[0, 0], (1, 1, 128))


def _stats(g1, g2):
    grid = (_B, _NT)
    return pl.pallas_call(
        _stats_body,
        grid=grid,
        in_specs=[
            pl.BlockSpec((1, 1, _SQ, _K, _D), lambda b, t: (b, t, 0, 0, 0)),
            pl.BlockSpec((1, 1, _SQ, _K, 16), lambda b, t: (b, t, 0, 0, 0)),
        ],
        out_specs=(
            pl.BlockSpec((1, 1, _SQ, _D), lambda b, t: (b, t, 0, 0)),
            pl.BlockSpec((1, 1, _SQ, 16), lambda b, t: (b, t, 0, 0)),
            pl.BlockSpec((1, 1, 128), lambda b, t: (b, 0, 0)),
        ),
        out_shape=(
            jax.ShapeDtypeStruct((_B, _NT, _SQ, _D), jnp.float32),
            jax.ShapeDtypeStruct((_B, _NT, _SQ, 16), jnp.float32),
            jax.ShapeDtypeStruct((_B, 1, 128), jnp.float32),
        ),
        scratch_shapes=[pltpu.VMEM((1, 1), jnp.float32)],
    )(g1, g2)


def _final_body(g1_ref, g2_ref, g3_ref, m1_ref, m2_ref, var_ref,
                a1_ref, a2_ref, b1_ref, b2_ref, out_ref):
    n1 = _S * _K * _CH - 1
    std = jnp.sqrt(var_ref[0, 0, 0] / n1)
    den = std + 1e-5
    g1 = g1_ref[0, 0]
    g2 = g2_ref[0, 0]
    m1 = m1_ref[0, 0]
    m2 = m2_ref[0, 0]
    p1 = (a1_ref[0][None, None, :] * ((g1 - m1[:, None]) / den)
          + b1_ref[0][None, None, :])
    p2 = (a2_ref[0][None, None, :] * ((g2 - m2[:, None]) / den)
          + b2_ref[0][None, None, :])
    g3 = g3_ref[0, 0]
    p3 = jnp.broadcast_to(g3[:, None, :], (_SQ, _K, _D))
    out_ref[0] = jnp.concatenate([p1, p2[:, :, :3], p3], axis=-1)


def _final(g1, g2, g3, m1, m2, var, a1, a2, b1, b2):
    grid = (_B, _NT)
    return pl.pallas_call(
        _final_body,
        grid=grid,
        in_specs=[
            pl.BlockSpec((1, 1, _SQ, _K, _D), lambda b, t: (b, t, 0, 0, 0)),
            pl.BlockSpec((1, 1, _SQ, _K, 16), lambda b, t: (b, t, 0, 0, 0)),
            pl.BlockSpec((1, 1, _SQ, _D), lambda b, t: (b, t, 0, 0)),
            pl.BlockSpec((1, 1, _SQ, _D), lambda b, t: (b, t, 0, 0)),
            pl.BlockSpec((1, 1, _SQ, 16), lambda b, t: (b, t, 0, 0)),
            pl.BlockSpec((1, 1, 128), lambda b, t: (b, 0, 0)),
            pl.BlockSpec((1, _D), lambda b, t: (0, 0)),
            pl.BlockSpec((1, 16), lambda b, t: (0, 0)),
            pl.BlockSpec((1, _D), lambda b, t: (0, 0)),
            pl.BlockSpec((1, 16), lambda b, t: (0, 0)),
        ],
        out_specs=pl.BlockSpec((1, _SQ, _K, _D * 2 + 3),
                               lambda b, t: (b, t, 0, 0)),
        out_shape=jax.ShapeDtypeStruct((_B, _S, _K, _D * 2 + 3), jnp.float32),
    )(g1, g2, g3, m1, m2, var, a1, a2, b1, b2)


def kernel(xyz, points, affine_alpha, affine_beta):
    new_xyz, fidx = _fps(xyz)

    xq = jnp.pad(jnp.transpose(xyz, (2, 0, 1)), ((0, 5), (0, 0), (0, 0)))
    xq = jnp.transpose(xq, (1, 0, 2))
    sq = jnp.sum(xyz * xyz, axis=-1).reshape(_B, 1, _N)
    nxq = jnp.pad(new_xyz, ((0, 0), (0, 0), (0, 5))).reshape(_B, _NT, _SQ, 8)
    idxg = _knn(xq, sq, nxq)

    t1 = points.reshape(_B * _N, _D)
    xyzf = xyz.reshape(_B * _N * 3)
    g1, g2, g3 = _gather(t1, xyzf, idxg.reshape(_NTOT), fidx.reshape(_NF))

    g1v = g1.reshape(_B, _NT, _SQ, _K, _D)
    g2v = g2.reshape(_B, _NT, _SQ, _K, 16)
    m1, m2, var = _stats(g1v, g2v)

    al = affine_alpha.reshape(_CH)
    be = affine_beta.reshape(_CH)
    a1 = al[:_D].reshape(1, _D)
    a2 = jnp.pad(al[_D:], (0, 13)).reshape(1, 16)
    b1 = be[:_D].reshape(1, _D)
    b2 = jnp.pad(be[_D:], (0, 13)).reshape(1, 16)

    g3v = g3.reshape(_B, _NT, _SQ, _D)
    out = _final(g1v, g2v, g3v, m1, m2, var, a1, a2, b1, b2)
    return new_xyz, out
